# baseline (speedup 1.0000x reference)
import functools

import jax
import jax.numpy as jnp
from jax import lax
from jax.experimental import pallas as pl
from jax.experimental.pallas import tpu as pltpu
from jax.experimental.pallas import tpu_sc as plsc

_LOG2E = 1.4426950408889634
_LN2 = 0.6931471805599453
_R = 16
_BC = 4
_BTC = 28
_NW = 32
_CR = 32
_E = _CR * 512


def _tc_body(lab_ref, x_ref, p_ref, n_ref, out_ref, acc_ref):
    i = pl.program_id(0)
    nb = pl.num_programs(0)

    @pl.when(i == 0)
    def _():
        acc_ref[...] = jnp.zeros_like(acc_ref)

    H = x_ref.shape[1]
    W = x_ref.shape[2]
    ss = [1.0 - 2.0 * lab_ref[b, 0, 0] for b in range(_BC)]

    def body(j, carry):
        asum, acnt = carry
        for b in range(_BC):
            x = x_ref[b, pl.ds(j * _R, _R), :]
            p = p_ref[b, pl.ds(j * _R, _R), :]
            n = n_ref[b, pl.ds(j * _R, _R), :]
            mask = (p > 0.5) & (n > 0.5)
            t = x * ss[b]
            u = jnp.exp2(jnp.abs(x) * (-_LOG2E))
            bce = jnp.maximum(t, 0.0) + _LN2 * jnp.log2(1.0 + u)
            asum = asum + jnp.where(mask, bce, 0.0)
            acnt = acnt + jnp.where(mask, 1.0, 0.0)
        return asum, acnt

    z = jnp.zeros((_R, W), jnp.float32)
    asum, acnt = jax.lax.fori_loop(0, H // _R, body, (z, z))
    acc_ref[0:1, :] += jnp.sum(asum, axis=0, keepdims=True)
    acc_ref[1:2, :] += jnp.sum(acnt, axis=0, keepdims=True)

    @pl.when(i == nb - 1)
    def _():
        out_ref[0, 0] = jnp.sum(acc_ref[0, :])
        out_ref[0, 1] = jnp.sum(acc_ref[1, :])


def _tc_part(x, p, n, lab, Btc, H, W):
    out = pl.pallas_call(
        _tc_body,
        grid=(Btc // _BC,),
        in_specs=[
            pl.BlockSpec((_BC, 1, 1), lambda i: (i, 0, 0), memory_space=pltpu.SMEM),
            pl.BlockSpec((_BC, H, W), lambda i: (i, 0, 0)),
            pl.BlockSpec((_BC, H, W), lambda i: (i, 0, 0)),
            pl.BlockSpec((_BC, H, W), lambda i: (i, 0, 0)),
        ],
        out_specs=pl.BlockSpec((1, 2), lambda i: (0, 0), memory_space=pltpu.SMEM),
        out_shape=jax.ShapeDtypeStruct((1, 2), jnp.float32),
        scratch_shapes=[
            pltpu.VMEM((2, W), jnp.float32),
        ],
    )(lab.reshape(-1, 1, 1), x, p, n)
    return out[0, 0], out[0, 1]


_P0 = 9.97503255e-06
_P1 = 0.999235484
_P2 = -0.490230723
_P3 = 0.285272681
_P4 = -0.131581825
_P5 = 0.0304490045
_U = 4


def _sc_compute_chunk(xb, pb, nb_, slot, sv, as0, ac0):
    def row_body(r, carry):
        def col_body(cb, carry2):
            as_, ac = carry2
            for j in range(_U):
                c = cb * (16 * _U) + j * 16
                xx = xb[slot, r, pl.ds(c, 16)]
                pp = pb[slot, r, pl.ds(c, 16)]
                nn = nb_[slot, r, pl.ds(c, 16)]
                mask = jnp.minimum(pp, nn) > 0.5
                t = xx * sv
                u = jnp.exp(jnp.minimum(xx, -xx))
                lg = _P0 + u * (_P1 + u * (_P2 + u * (_P3 + u * (_P4 + u * _P5))))
                bce = jnp.maximum(t, 0.0) + lg
                as_ = as_ + jnp.where(mask, bce, 0.0)
                ac = ac + jnp.where(mask, 1.0, 0.0)
            return as_, ac

        return lax.fori_loop(0, 512 // (16 * _U), col_body, carry)

    return lax.fori_loop(0, _CR, row_body, (as0, ac0))


def _make_sc_part(base_elem, elems_per_worker):
    nchunks = elems_per_worker // _E
    mesh = plsc.VectorSubcoreMesh(core_axis_name="c", subcore_axis_name="s")

    @functools.partial(
        pl.kernel,
        out_type=[
            jax.ShapeDtypeStruct((_NW, 16), jnp.float32),
            jax.ShapeDtypeStruct((_NW, 16), jnp.float32),
        ],
        mesh=mesh,
        scratch_types=[
            pltpu.VMEM((2, _CR, 512), jnp.float32),
            pltpu.VMEM((2, _CR, 512), jnp.float32),
            pltpu.VMEM((2, _CR, 512), jnp.float32),
            pltpu.VMEM((2, 16), jnp.float32),
            pltpu.VMEM((2, 16), jnp.float32),
            pltpu.SemaphoreType.DMA,
            pltpu.SemaphoreType.DMA,
        ],
    )
    def sc_kernel(x_hbm, p_hbm, n_hbm, s_hbm, osum, ocnt, xb, pb, nb_, sb, accs, semA, semB):
        wid = lax.axis_index("s") * 2 + lax.axis_index("c")
        rpw = elems_per_worker // 512
        base = base_elem // 512 + wid * rpw
        sems = [semA, semB]

        def start(g, slot):
            r0 = base + g * _CR
            f = wid * nchunks + g
            return [
                pltpu.async_copy(x_hbm.at[pl.ds(r0, _CR), :], xb.at[slot], sems[slot]),
                pltpu.async_copy(p_hbm.at[pl.ds(r0, _CR), :], pb.at[slot], sems[slot]),
                pltpu.async_copy(n_hbm.at[pl.ds(r0, _CR), :], nb_.at[slot], sems[slot]),
                pltpu.async_copy(s_hbm.at[f], sb.at[slot], sems[slot]),
            ]

        handles = start(0, 0)
        as_ = jnp.zeros((16,), jnp.float32)
        ac = jnp.zeros((16,), jnp.float32)
        for g in range(nchunks):
            slot = g % 2
            nxt = None
            if g + 1 < nchunks:
                nxt = start(g + 1, 1 - slot)
            for h in handles:
                h.wait()
            sv = sb[slot, :]
            as_, ac = _sc_compute_chunk(xb, pb, nb_, slot, sv, as_, ac)
            handles = nxt

        accs[0, :] = as_
        accs[1, :] = ac
        pltpu.sync_copy(accs.at[0], osum.at[wid])
        pltpu.sync_copy(accs.at[1], ocnt.at[wid])

    return sc_kernel


def kernel(cancer_logits, label, prostate_mask, needle_mask):
    B, C, H, W = cancer_logits.shape
    x = cancer_logits.reshape(B, H, W)
    p = prostate_mask.reshape(B, H, W)
    n = needle_mask.reshape(B, H, W)

    total = jnp.float32(0.0)
    count = jnp.float32(0.0)

    if _BTC > 0:
        ts, tc = _tc_part(x, p, n, label, _BTC, H, W)
        total = total + ts
        count = count + tc

    Bsc = B - _BTC
    if Bsc > 0:
        base_elem = _BTC * H * W
        epw = Bsc * H * W // _NW
        nchunks = epw // _E
        signs = 1.0 - 2.0 * label[_BTC:]
        s_chunks = jnp.broadcast_to(
            jnp.repeat(signs, H // _CR)[:, None], (_NW * nchunks, 16)
        )
        sc = _make_sc_part(base_elem, epw)
        osum, ocnt = sc(
            x.reshape(B * H, W), p.reshape(B * H, W), n.reshape(B * H, W), s_chunks
        )
        total = total + jnp.sum(osum)
        count = count + jnp.sum(ocnt)

    return total / count


# speedup vs baseline: 1.5311x; 1.5311x over previous
import functools

import jax
import jax.numpy as jnp
from jax import lax
from jax.experimental import pallas as pl
from jax.experimental.pallas import tpu as pltpu
from jax.experimental.pallas import tpu_sc as plsc

_LOG2E = 1.4426950408889634
_LN2 = 0.6931471805599453
_R = 16
_BC = 4
_BTC = 32
_NW = 32
_CR = 32
_E = _CR * 512


def _tc_body(lab_ref, x_ref, p_ref, n_ref, out_ref, acc_ref):
    i = pl.program_id(0)
    nb = pl.num_programs(0)

    @pl.when(i == 0)
    def _():
        acc_ref[...] = jnp.zeros_like(acc_ref)

    H = x_ref.shape[1]
    W = x_ref.shape[2]
    ss = [1.0 - 2.0 * lab_ref[b, 0, 0] for b in range(_BC)]

    def body(j, carry):
        asum, acnt = carry
        for b in range(_BC):
            x = x_ref[b, pl.ds(j * _R, _R), :]
            p = p_ref[b, pl.ds(j * _R, _R), :]
            n = n_ref[b, pl.ds(j * _R, _R), :]
            mask = (p > 0.5) & (n > 0.5)
            t = x * ss[b]
            u = jnp.exp2(jnp.abs(x) * (-_LOG2E))
            bce = jnp.maximum(t, 0.0) + _LN2 * jnp.log2(1.0 + u)
            asum = asum + jnp.where(mask, bce, 0.0)
            acnt = acnt + jnp.where(mask, 1.0, 0.0)
        return asum, acnt

    z = jnp.zeros((_R, W), jnp.float32)
    asum, acnt = jax.lax.fori_loop(0, H // _R, body, (z, z))
    acc_ref[0:1, :] += jnp.sum(asum, axis=0, keepdims=True)
    acc_ref[1:2, :] += jnp.sum(acnt, axis=0, keepdims=True)

    @pl.when(i == nb - 1)
    def _():
        out_ref[0, 0] = jnp.sum(acc_ref[0, :])
        out_ref[0, 1] = jnp.sum(acc_ref[1, :])


def _tc_part(x, p, n, lab, Btc, H, W):
    out = pl.pallas_call(
        _tc_body,
        grid=(Btc // _BC,),
        in_specs=[
            pl.BlockSpec((_BC, 1, 1), lambda i: (i, 0, 0), memory_space=pltpu.SMEM),
            pl.BlockSpec((_BC, H, W), lambda i: (i, 0, 0)),
            pl.BlockSpec((_BC, H, W), lambda i: (i, 0, 0)),
            pl.BlockSpec((_BC, H, W), lambda i: (i, 0, 0)),
        ],
        out_specs=pl.BlockSpec((1, 2), lambda i: (0, 0), memory_space=pltpu.SMEM),
        out_shape=jax.ShapeDtypeStruct((1, 2), jnp.float32),
        scratch_shapes=[
            pltpu.VMEM((2, W), jnp.float32),
        ],
    )(lab.reshape(-1, 1, 1), x, p, n)
    return out[0, 0], out[0, 1]


_P0 = 9.97503255e-06
_P1 = 0.999235484
_P2 = -0.490230723
_P3 = 0.285272681
_P4 = -0.131581825
_P5 = 0.0304490045
_U = 4


def _sc_compute_chunk(xb, pb, nb_, slot, sv, as0, ac0):
    def row_body(r, carry):
        def col_body(cb, carry2):
            as_, ac = carry2
            for j in range(_U):
                c = cb * (16 * _U) + j * 16
                xx = xb[slot, r, pl.ds(c, 16)]
                pp = pb[slot, r, pl.ds(c, 16)]
                nn = nb_[slot, r, pl.ds(c, 16)]
                mask = jnp.minimum(pp, nn) > 0.5
                t = xx * sv
                u = jnp.exp(jnp.minimum(xx, -xx))
                lg = _P0 + u * (_P1 + u * (_P2 + u * (_P3 + u * (_P4 + u * _P5))))
                bce = jnp.maximum(t, 0.0) + lg
                as_ = as_ + jnp.where(mask, bce, 0.0)
                ac = ac + jnp.where(mask, 1.0, 0.0)
            return as_, ac

        return lax.fori_loop(0, 512 // (16 * _U), col_body, carry)

    return lax.fori_loop(0, _CR, row_body, (as0, ac0))


def _make_sc_part(base_elem, elems_per_worker):
    nchunks = elems_per_worker // _E
    mesh = plsc.VectorSubcoreMesh(core_axis_name="c", subcore_axis_name="s")

    @functools.partial(
        pl.kernel,
        out_type=[
            jax.ShapeDtypeStruct((_NW, 16), jnp.float32),
            jax.ShapeDtypeStruct((_NW, 16), jnp.float32),
        ],
        mesh=mesh,
        scratch_types=[
            pltpu.VMEM((2, _CR, 512), jnp.float32),
            pltpu.VMEM((2, _CR, 512), jnp.float32),
            pltpu.VMEM((2, _CR, 512), jnp.float32),
            pltpu.VMEM((2, 16), jnp.float32),
            pltpu.VMEM((2, 16), jnp.float32),
            pltpu.SemaphoreType.DMA,
            pltpu.SemaphoreType.DMA,
        ],
    )
    def sc_kernel(x_hbm, p_hbm, n_hbm, s_hbm, osum, ocnt, xb, pb, nb_, sb, accs, semA, semB):
        wid = lax.axis_index("s") * 2 + lax.axis_index("c")
        rpw = elems_per_worker // 512
        base = base_elem // 512 + wid * rpw
        sems = [semA, semB]

        def start(g, slot):
            r0 = base + g * _CR
            f = wid * nchunks + g
            return [
                pltpu.async_copy(x_hbm.at[pl.ds(r0, _CR), :], xb.at[slot], sems[slot]),
                pltpu.async_copy(p_hbm.at[pl.ds(r0, _CR), :], pb.at[slot], sems[slot]),
                pltpu.async_copy(n_hbm.at[pl.ds(r0, _CR), :], nb_.at[slot], sems[slot]),
                pltpu.async_copy(s_hbm.at[f], sb.at[slot], sems[slot]),
            ]

        handles = start(0, 0)
        as_ = jnp.zeros((16,), jnp.float32)
        ac = jnp.zeros((16,), jnp.float32)
        for g in range(nchunks):
            slot = g % 2
            nxt = None
            if g + 1 < nchunks:
                nxt = start(g + 1, 1 - slot)
            for h in handles:
                h.wait()
            sv = sb[slot, :]
            as_, ac = _sc_compute_chunk(xb, pb, nb_, slot, sv, as_, ac)
            handles = nxt

        accs[0, :] = as_
        accs[1, :] = ac
        pltpu.sync_copy(accs.at[0], osum.at[wid])
        pltpu.sync_copy(accs.at[1], ocnt.at[wid])

    return sc_kernel


def kernel(cancer_logits, label, prostate_mask, needle_mask):
    B, C, H, W = cancer_logits.shape
    x = cancer_logits.reshape(B, H, W)
    p = prostate_mask.reshape(B, H, W)
    n = needle_mask.reshape(B, H, W)

    total = jnp.float32(0.0)
    count = jnp.float32(0.0)

    if _BTC > 0:
        ts, tc = _tc_part(x, p, n, label, _BTC, H, W)
        total = total + ts
        count = count + tc

    Bsc = B - _BTC
    if Bsc > 0:
        base_elem = _BTC * H * W
        epw = Bsc * H * W // _NW
        nchunks = epw // _E
        signs = 1.0 - 2.0 * label[_BTC:]
        s_chunks = jnp.broadcast_to(
            jnp.repeat(signs, H // _CR)[:, None], (_NW * nchunks, 16)
        )
        sc = _make_sc_part(base_elem, epw)
        osum, ocnt = sc(
            x.reshape(B * H, W), p.reshape(B * H, W), n.reshape(B * H, W), s_chunks
        )
        total = total + jnp.sum(osum)
        count = count + jnp.sum(ocnt)

    return total / count


# speedup vs baseline: 1.6619x; 1.0854x over previous
import jax
import jax.numpy as jnp
from jax.experimental import pallas as pl
from jax.experimental.pallas import tpu as pltpu

_LOG2E = 1.4426950408889634
_LN2 = 0.6931471805599453
_R = 16
_BC = 4


def _tc_body(lab_ref, x_ref, p_ref, n_ref, out_ref, acc_ref):
    i = pl.program_id(0)
    nb = pl.num_programs(0)

    @pl.when(i == 0)
    def _():
        acc_ref[...] = jnp.zeros_like(acc_ref)

    H = x_ref.shape[1]
    W = x_ref.shape[2]
    ss = [1.0 - 2.0 * lab_ref[b, 0, 0] for b in range(_BC)]

    def body(j, carry):
        asum, acnt = carry
        for b in range(_BC):
            x = x_ref[b, pl.ds(j * _R, _R), :]
            p = p_ref[b, pl.ds(j * _R, _R), :]
            n = n_ref[b, pl.ds(j * _R, _R), :]
            mask = (p > 0.5) & (n > 0.5)
            t = x * ss[b]
            u = jnp.exp2(jnp.abs(x) * (-_LOG2E))
            bce = jnp.maximum(t, 0.0) + _LN2 * jnp.log2(1.0 + u)
            asum = asum + jnp.where(mask, bce, 0.0)
            acnt = acnt + jnp.where(mask, 1.0, 0.0)
        return asum, acnt

    z = jnp.zeros((_R, W), jnp.float32)
    asum, acnt = jax.lax.fori_loop(0, H // _R, body, (z, z))
    acc_ref[0:1, :] += jnp.sum(asum, axis=0, keepdims=True)
    acc_ref[1:2, :] += jnp.sum(acnt, axis=0, keepdims=True)

    @pl.when(i == nb - 1)
    def _():
        out_ref[0, 0] = jnp.sum(acc_ref[0, :]) / jnp.sum(acc_ref[1, :])


def kernel(cancer_logits, label, prostate_mask, needle_mask):
    B, C, H, W = cancer_logits.shape
    x = cancer_logits.reshape(B, H, W)
    p = prostate_mask.reshape(B, H, W)
    n = needle_mask.reshape(B, H, W)
    lab = label.reshape(B, 1, 1)

    out = pl.pallas_call(
        _tc_body,
        grid=(B // _BC,),
        in_specs=[
            pl.BlockSpec((_BC, 1, 1), lambda i: (i, 0, 0), memory_space=pltpu.SMEM),
            pl.BlockSpec((_BC, H, W), lambda i: (i, 0, 0)),
            pl.BlockSpec((_BC, H, W), lambda i: (i, 0, 0)),
            pl.BlockSpec((_BC, H, W), lambda i: (i, 0, 0)),
        ],
        out_specs=pl.BlockSpec((1, 1), lambda i: (0, 0), memory_space=pltpu.SMEM),
        out_shape=jax.ShapeDtypeStruct((1, 1), jnp.float32),
        scratch_shapes=[
            pltpu.VMEM((2, W), jnp.float32),
        ],
    )(lab, x, p, n)
    return out[0, 0]
